# baseline (speedup 1.0000x reference)
import math

import jax
import jax.numpy as jnp
from jax import lax
from jax.experimental import pallas as pl
from jax.experimental.pallas import tpu as pltpu
from jax.experimental.pallas import tpu_sc as plsc

_BETA = 0.9
_LN_BETA = math.log(_BETA)

_N, _A = 16384, 26
_NC, _NS = 2, 16
_NW = _NC * _NS
_CPW = _N // _NW
_EPW = _CPW * _A
_L = 16
_NCH = 4
_CHE = _EPW // _NCH
_UNROLL = 4
_CSTEP = _CHE // (_L * _UNROLL)


def _fair_scaler_body(attr_hbm, ms_hbm, out_hbm, idx_v, vals_v,
                      sem_io, sem_g0, sem_g1):
    wid = lax.axis_index("s") * _NC + lax.axis_index("c")
    c0 = wid * _CPW
    copies = [
        pltpu.make_async_copy(
            attr_hbm.at[a, pl.ds(c0, _CPW)],
            idx_v.at[pl.ds(a * _CPW, _CPW)],
            sem_io,
        )
        for a in range(_A)
    ]
    for c in copies:
        c.start()
    for c in copies:
        c.wait()

    sems = (sem_g0, sem_g1)
    gathers = [
        pltpu.make_async_copy(
            ms_hbm.at[idx_v.at[pl.ds(c * _CHE, _CHE)]],
            vals_v.at[pl.ds(c * _CHE, _CHE)],
            sems[c % 2],
        )
        for c in range(_NCH)
    ]
    gathers[0].start()
    gathers[1].start()
    for c in range(_NCH):
        gathers[c].wait()
        if c + 2 < _NCH:
            gathers[c + 2].start()

        def step(k, carry, base=c * _CHE):
            for j in range(_UNROLL):
                o = base + k * (_L * _UNROLL) + j * _L
                n = vals_v[pl.ds(o, _L)]
                w = (1.0 - _BETA) / (1.0 - jnp.exp(n * _LN_BETA))
                vals_v[pl.ds(o, _L)] = w
            return carry

        lax.fori_loop(0, _CSTEP, step, 0)

    copies = [
        pltpu.make_async_copy(
            vals_v.at[pl.ds(a * _CPW, _CPW)],
            out_hbm.at[a, pl.ds(c0, _CPW)],
            sem_io,
        )
        for a in range(_A)
    ]
    for c in copies:
        c.start()
    for c in copies:
        c.wait()


_sc_call = pl.kernel(
    _fair_scaler_body,
    mesh=plsc.VectorSubcoreMesh(core_axis_name="c", subcore_axis_name="s"),
    out_type=jax.ShapeDtypeStruct((_A, _N), jnp.float32),
    scratch_types=[
        pltpu.VMEM((_EPW,), jnp.int32),
        pltpu.VMEM((_EPW,), jnp.float32),
        pltpu.SemaphoreType.DMA,
        pltpu.SemaphoreType.DMA,
        pltpu.SemaphoreType.DMA,
    ],
)


def kernel(attr, metric_scores):
    return _sc_call(attr.T, metric_scores).T


# speedup vs baseline: 1.0368x; 1.0368x over previous
import math

import jax
import jax.numpy as jnp
from jax import lax
from jax.experimental import pallas as pl
from jax.experimental.pallas import tpu as pltpu
from jax.experimental.pallas import tpu_sc as plsc

_BETA = 0.9
_LN_BETA = math.log(_BETA)

_N, _A = 16384, 26
_NC, _NS = 2, 16
_NW = _NC * _NS
_CPW = _N // _NW
_EPW = _CPW * _A
_L = 16
_NCH = 16
_CHE = _EPW // _NCH
_UNROLL = 4
_CSTEP = _CHE // (_L * _UNROLL)


def _fair_scaler_body(attr_hbm, ms_hbm, out_hbm, idx_v, vals_v,
                      sem_io, sem_g0, sem_g1):
    wid = lax.axis_index("s") * _NC + lax.axis_index("c")
    c0 = wid * _CPW
    copies = [
        pltpu.make_async_copy(
            attr_hbm.at[a, pl.ds(c0, _CPW)],
            idx_v.at[pl.ds(a * _CPW, _CPW)],
            sem_io,
        )
        for a in range(_A)
    ]
    for c in copies:
        c.start()
    for c in copies:
        c.wait()

    sems = (sem_g0, sem_g1)
    gathers = [
        pltpu.make_async_copy(
            ms_hbm.at[idx_v.at[pl.ds(c * _CHE, _CHE)]],
            vals_v.at[pl.ds(c * _CHE, _CHE)],
            sems[c % 2],
        )
        for c in range(_NCH)
    ]
    gathers[0].start()
    gathers[1].start()
    for c in range(_NCH):
        gathers[c].wait()
        if c + 2 < _NCH:
            gathers[c + 2].start()

        def step(k, carry, base=c * _CHE):
            for j in range(_UNROLL):
                o = base + k * (_L * _UNROLL) + j * _L
                n = vals_v[pl.ds(o, _L)]
                w = (1.0 - _BETA) / (1.0 - jnp.exp(n * _LN_BETA))
                vals_v[pl.ds(o, _L)] = w
            return carry

        lax.fori_loop(0, _CSTEP, step, 0)

    copies = [
        pltpu.make_async_copy(
            vals_v.at[pl.ds(a * _CPW, _CPW)],
            out_hbm.at[a, pl.ds(c0, _CPW)],
            sem_io,
        )
        for a in range(_A)
    ]
    for c in copies:
        c.start()
    for c in copies:
        c.wait()


_sc_call = pl.kernel(
    _fair_scaler_body,
    mesh=plsc.VectorSubcoreMesh(core_axis_name="c", subcore_axis_name="s"),
    out_type=jax.ShapeDtypeStruct((_A, _N), jnp.float32),
    scratch_types=[
        pltpu.VMEM((_EPW,), jnp.int32),
        pltpu.VMEM((_EPW,), jnp.float32),
        pltpu.SemaphoreType.DMA,
        pltpu.SemaphoreType.DMA,
        pltpu.SemaphoreType.DMA,
    ],
)


def kernel(attr, metric_scores):
    return _sc_call(attr.T, metric_scores).T


# speedup vs baseline: 1.0392x; 1.0023x over previous
import math

import jax
import jax.numpy as jnp
from jax import lax
from jax.experimental import pallas as pl
from jax.experimental.pallas import tpu as pltpu
from jax.experimental.pallas import tpu_sc as plsc

_BETA = 0.9
_LN_BETA = math.log(_BETA)

_N, _A = 16384, 26
_NC, _NS = 2, 16
_NW = _NC * _NS
_CPW = _N // _NW
_EPW = _CPW * _A
_L = 16
_NCH = 16
_CHE = _EPW // _NCH
_UNROLL = 4
_CSTEP = _CHE // (_L * _UNROLL)


def _fair_scaler_body(attr_hbm, ms_hbm, out_hbm, idx_v, vals_v,
                      sem_io, sem_g0, sem_g1, sem_g2):
    wid = lax.axis_index("s") * _NC + lax.axis_index("c")
    c0 = wid * _CPW
    copies = [
        pltpu.make_async_copy(
            attr_hbm.at[a, pl.ds(c0, _CPW)],
            idx_v.at[pl.ds(a * _CPW, _CPW)],
            sem_io,
        )
        for a in range(_A)
    ]
    for c in copies:
        c.start()
    for c in copies:
        c.wait()

    sems = (sem_g0, sem_g1, sem_g2)
    gathers = [
        pltpu.make_async_copy(
            ms_hbm.at[idx_v.at[pl.ds(c * _CHE, _CHE)]],
            vals_v.at[pl.ds(c * _CHE, _CHE)],
            sems[c % 3],
        )
        for c in range(_NCH)
    ]
    gathers[0].start()
    gathers[1].start()
    gathers[2].start()
    for c in range(_NCH):
        gathers[c].wait()
        if c + 3 < _NCH:
            gathers[c + 3].start()

        def step(k, carry, base=c * _CHE):
            for j in range(_UNROLL):
                o = base + k * (_L * _UNROLL) + j * _L
                n = vals_v[pl.ds(o, _L)]
                w = (1.0 - _BETA) / (1.0 - jnp.exp(n * _LN_BETA))
                vals_v[pl.ds(o, _L)] = w
            return carry

        lax.fori_loop(0, _CSTEP, step, 0)

    copies = [
        pltpu.make_async_copy(
            vals_v.at[pl.ds(a * _CPW, _CPW)],
            out_hbm.at[a, pl.ds(c0, _CPW)],
            sem_io,
        )
        for a in range(_A)
    ]
    for c in copies:
        c.start()
    for c in copies:
        c.wait()


_sc_call = pl.kernel(
    _fair_scaler_body,
    mesh=plsc.VectorSubcoreMesh(core_axis_name="c", subcore_axis_name="s"),
    out_type=jax.ShapeDtypeStruct((_A, _N), jnp.float32),
    scratch_types=[
        pltpu.VMEM((_EPW,), jnp.int32),
        pltpu.VMEM((_EPW,), jnp.float32),
        pltpu.SemaphoreType.DMA,
        pltpu.SemaphoreType.DMA,
        pltpu.SemaphoreType.DMA,
        pltpu.SemaphoreType.DMA,
    ],
)


def kernel(attr, metric_scores):
    return _sc_call(attr.T, metric_scores).T


# speedup vs baseline: 1.0452x; 1.0058x over previous
import math

import jax
import jax.numpy as jnp
from jax import lax
from jax.experimental import pallas as pl
from jax.experimental.pallas import tpu as pltpu
from jax.experimental.pallas import tpu_sc as plsc

_BETA = 0.9
_LN_BETA = math.log(_BETA)

_N, _A = 16384, 26
_NC, _NS = 2, 16
_NW = _NC * _NS
_CPW = _N // _NW
_EPW = _CPW * _A
_L = 16
_NCH = 8
_CHE = _EPW // _NCH
_UNROLL = 8
_CSTEP = _CHE // (_L * _UNROLL)


def _fair_scaler_body(attr_hbm, ms_hbm, out_hbm, idx_v, vals_v,
                      sem_io, sem_g0, sem_g1, sem_g2):
    wid = lax.axis_index("s") * _NC + lax.axis_index("c")
    c0 = wid * _CPW
    copies = [
        pltpu.make_async_copy(
            attr_hbm.at[a, pl.ds(c0, _CPW)],
            idx_v.at[pl.ds(a * _CPW, _CPW)],
            sem_io,
        )
        for a in range(_A)
    ]
    for c in copies:
        c.start()
    for c in copies:
        c.wait()

    sems = (sem_g0, sem_g1, sem_g2)
    gathers = [
        pltpu.make_async_copy(
            ms_hbm.at[idx_v.at[pl.ds(c * _CHE, _CHE)]],
            vals_v.at[pl.ds(c * _CHE, _CHE)],
            sems[c % 3],
        )
        for c in range(_NCH)
    ]
    gathers[0].start()
    gathers[1].start()
    gathers[2].start()
    for c in range(_NCH):
        gathers[c].wait()
        if c + 3 < _NCH:
            gathers[c + 3].start()

        def step(k, carry, base=c * _CHE):
            for j in range(_UNROLL):
                o = base + k * (_L * _UNROLL) + j * _L
                n = vals_v[pl.ds(o, _L)]
                w = (1.0 - _BETA) / (1.0 - jnp.exp(n * _LN_BETA))
                vals_v[pl.ds(o, _L)] = w
            return carry

        lax.fori_loop(0, _CSTEP, step, 0)

    copies = [
        pltpu.make_async_copy(
            vals_v.at[pl.ds(a * _CPW, _CPW)],
            out_hbm.at[a, pl.ds(c0, _CPW)],
            sem_io,
        )
        for a in range(_A)
    ]
    for c in copies:
        c.start()
    for c in copies:
        c.wait()


_sc_call = pl.kernel(
    _fair_scaler_body,
    mesh=plsc.VectorSubcoreMesh(core_axis_name="c", subcore_axis_name="s"),
    out_type=jax.ShapeDtypeStruct((_A, _N), jnp.float32),
    scratch_types=[
        pltpu.VMEM((_EPW,), jnp.int32),
        pltpu.VMEM((_EPW,), jnp.float32),
        pltpu.SemaphoreType.DMA,
        pltpu.SemaphoreType.DMA,
        pltpu.SemaphoreType.DMA,
        pltpu.SemaphoreType.DMA,
    ],
)


def kernel(attr, metric_scores):
    return _sc_call(attr.T, metric_scores).T


# speedup vs baseline: 1.0520x; 1.0065x over previous
import math

import jax
import jax.numpy as jnp
from jax import lax
from jax.experimental import pallas as pl
from jax.experimental.pallas import tpu as pltpu
from jax.experimental.pallas import tpu_sc as plsc

_BETA = 0.9
_LN_BETA = math.log(_BETA)

_N, _A = 16384, 26
_NC, _NS = 2, 16
_NW = _NC * _NS
_CPW = _N // _NW
_EPW = _CPW * _A
_L = 16
_NCH = 8
_CHE = _EPW // _NCH
_UNROLL = 8
_CSTEP = _CHE // (_L * _UNROLL)


def _fair_scaler_body(attr_hbm, ms_hbm, out_hbm, idx_v, vals_v,
                      sem_io, sem_g0, sem_g1, sem_g2):
    wid = lax.axis_index("s") * _NC + lax.axis_index("c")
    c0 = wid * _CPW
    copies = [
        pltpu.make_async_copy(
            attr_hbm.at[a, pl.ds(c0, _CPW)],
            idx_v.at[pl.ds(a * _CPW, _CPW)],
            sem_io,
        )
        for a in range(_A)
    ]
    for c in copies:
        c.start()
    for c in copies:
        c.wait()

    sems = (sem_g0, sem_g1, sem_g2)
    gathers = [
        pltpu.make_async_copy(
            ms_hbm.at[idx_v.at[pl.ds(c * _CHE, _CHE)]],
            vals_v.at[pl.ds(c * _CHE, _CHE)],
            sems[c % 3],
        )
        for c in range(_NCH)
    ]
    gathers[0].start()
    gathers[1].start()
    gathers[2].start()
    for c in range(_NCH):
        gathers[c].wait()
        if c + 3 < _NCH:
            gathers[c + 3].start()

        @plsc.parallel_loop(c * _CHE, (c + 1) * _CHE, step=_L,
                            unroll=_UNROLL)
        def _(o):
            n = vals_v[pl.ds(o, _L)]
            w = (1.0 - _BETA) / (1.0 - jnp.exp(n * _LN_BETA))
            vals_v[pl.ds(o, _L)] = w

    copies = [
        pltpu.make_async_copy(
            vals_v.at[pl.ds(a * _CPW, _CPW)],
            out_hbm.at[a, pl.ds(c0, _CPW)],
            sem_io,
        )
        for a in range(_A)
    ]
    for c in copies:
        c.start()
    for c in copies:
        c.wait()


_sc_call = pl.kernel(
    _fair_scaler_body,
    mesh=plsc.VectorSubcoreMesh(core_axis_name="c", subcore_axis_name="s"),
    out_type=jax.ShapeDtypeStruct((_A, _N), jnp.float32),
    scratch_types=[
        pltpu.VMEM((_EPW,), jnp.int32),
        pltpu.VMEM((_EPW,), jnp.float32),
        pltpu.SemaphoreType.DMA,
        pltpu.SemaphoreType.DMA,
        pltpu.SemaphoreType.DMA,
        pltpu.SemaphoreType.DMA,
    ],
)


def kernel(attr, metric_scores):
    return _sc_call(attr.T, metric_scores).T


# speedup vs baseline: 1.0596x; 1.0072x over previous
import math

import jax
import jax.numpy as jnp
from jax import lax
from jax.experimental import pallas as pl
from jax.experimental.pallas import tpu as pltpu
from jax.experimental.pallas import tpu_sc as plsc

_BETA = 0.9
_LN_BETA = math.log(_BETA)

_N, _A = 16384, 26
_NC, _NS = 2, 16
_NW = _NC * _NS
_CPW = _N // _NW
_EPW = _CPW * _A
_L = 16
_NCH = 8
_CHE = _EPW // _NCH
_UNROLL = 8
_CSTEP = _CHE // (_L * _UNROLL)


def _fair_scaler_body(attr_hbm, ms_hbm, out_hbm, idx_v, vals_v,
                      sem_io, sem_g0, sem_g1, sem_g2):
    wid = lax.axis_index("s") * _NC + lax.axis_index("c")
    c0 = wid * _CPW
    copies = [
        pltpu.make_async_copy(
            attr_hbm.at[a, pl.ds(c0, _CPW)],
            idx_v.at[pl.ds(a * _CPW, _CPW)],
            sem_io,
        )
        for a in range(_A)
    ]
    for c in copies:
        c.start()
    for c in copies:
        c.wait()

    sems = (sem_g0, sem_g1, sem_g2)
    gathers = [
        pltpu.make_async_copy(
            ms_hbm.at[idx_v.at[pl.ds(c * _CHE, _CHE)]],
            vals_v.at[pl.ds(c * _CHE, _CHE)],
            sems[c % 3],
        )
        for c in range(_NCH)
    ]
    def out_copy(a):
        return pltpu.make_async_copy(
            vals_v.at[pl.ds(a * _CPW, _CPW)],
            out_hbm.at[a, pl.ds(c0, _CPW)],
            sem_io,
        )

    gathers[0].start()
    gathers[1].start()
    gathers[2].start()
    rows_done = 0
    for c in range(_NCH):
        gathers[c].wait()
        if c + 3 < _NCH:
            gathers[c + 3].start()

        @plsc.parallel_loop(c * _CHE, (c + 1) * _CHE, step=_L,
                            unroll=_UNROLL)
        def _(o):
            n = vals_v[pl.ds(o, _L)]
            w = (1.0 - _BETA) / (1.0 - jnp.exp(n * _LN_BETA))
            vals_v[pl.ds(o, _L)] = w

        done = ((c + 1) * _CHE) // _CPW
        for a in range(rows_done, done):
            out_copy(a).start()
        rows_done = done

    for a in range(_A):
        out_copy(a).wait()


_sc_call = pl.kernel(
    _fair_scaler_body,
    mesh=plsc.VectorSubcoreMesh(core_axis_name="c", subcore_axis_name="s"),
    out_type=jax.ShapeDtypeStruct((_A, _N), jnp.float32),
    scratch_types=[
        pltpu.VMEM((_EPW,), jnp.int32),
        pltpu.VMEM((_EPW,), jnp.float32),
        pltpu.SemaphoreType.DMA,
        pltpu.SemaphoreType.DMA,
        pltpu.SemaphoreType.DMA,
        pltpu.SemaphoreType.DMA,
    ],
)


def kernel(attr, metric_scores):
    return _sc_call(attr.T, metric_scores).T
